# baseline (speedup 1.0000x reference)
import functools

import jax
import jax.numpy as jnp
from jax import lax
from jax.experimental import pallas as pl
from jax.experimental.pallas import tpu as pltpu
from jax.experimental.pallas import tpu_sc as plsc

K = 8192
D = 64
BETA = 0.25
TILE_M = 512
CHUNK = 2048

_SC_INFO = plsc.get_sparse_core_info()
_NC, _NS = _SC_INFO.num_cores, _SC_INFO.num_subcores
_NW = _NC * _NS


def _dist_body(z_ref, zsq_ref, c2_ref, csq_ref, k_ref, lp_ref):
    z = z_ref[...]
    dot2 = jax.lax.dot_general(
        z, c2_ref[...], (((1,), (1,)), ((), ())),
        preferred_element_type=jnp.float32)
    zsq = zsq_ref[...]
    csq = csq_ref[...]

    nch = K // CHUNK
    acc_v = cidx = minval = None
    for t in range(nch):
        sl = slice(t * CHUNK, (t + 1) * CHUNK)
        m = jnp.min((zsq - dot2[:, sl]) + csq[:, sl], axis=1, keepdims=True)
        if t == 0:
            minval = m
            cidx = jnp.zeros((TILE_M, 1), jnp.int32)
            acc_v = m.astype(jnp.bfloat16).astype(jnp.float32)
        else:
            upd = m < acc_v
            cidx = jnp.where(upd, t, cidx)
            minval = jnp.where(upd, m, minval)
            acc_v = jnp.where(upd, m, acc_v).astype(jnp.bfloat16).astype(jnp.float32)

    dot2sel = dot2[:, :CHUNK]
    csqsel = jnp.broadcast_to(csq[:, :CHUNK], (TILE_M, CHUNK))
    for t in range(1, nch):
        sel = cidx == t
        dot2sel = jnp.where(sel, dot2[:, t * CHUNK:(t + 1) * CHUNK], dot2sel)
        csqsel = jnp.where(sel, csq[:, t * CHUNK:(t + 1) * CHUNK], csqsel)
    dsel = (zsq - dot2sel) + csqsel
    iota_l = jax.lax.broadcasted_iota(jnp.int32, (TILE_M, CHUNK), 1)
    lidx = jnp.min(jnp.where(dsel == minval, iota_l, K), axis=1)
    k_ref[0, 0, :] = lidx + cidx[:, 0] * CHUNK
    lp_ref[...] = jnp.sum(minval).reshape(1, 1, 1)


_DPAD = 128


def _make_sc_gather(n_tokens):
    b_per_w = n_tokens // _NW
    mesh = plsc.VectorSubcoreMesh(core_axis_name="c", subcore_axis_name="s")

    @functools.partial(
        pl.kernel, mesh=mesh,
        out_type=[
            jax.ShapeDtypeStruct((n_tokens, _DPAD), jnp.float32),
            jax.ShapeDtypeStruct((_NC, K), jnp.int32),
        ],
        scratch_types=[
            pltpu.VMEM((b_per_w,), jnp.int32),
            pltpu.VMEM((b_per_w, _DPAD), jnp.float32),
            pltpu.VMEM((b_per_w,), jnp.int32),
            pltpu.VMEM((K,), jnp.int32),
            pltpu.VMEM_SHARED((K,), jnp.int32),
            pltpu.SemaphoreType.DMA,
        ],
    )
    def sc_gather(table_hbm, idx_hbm, zeros_hbm, ones_hbm,
                  zq_hbm, counts_hbm,
                  idx_v, rows_v, ones_v, cnt_v, shared, sem):
        cid = lax.axis_index("c")
        sid = lax.axis_index("s")
        wid = sid * _NC + cid
        base = wid * b_per_w
        pltpu.sync_copy(idx_hbm.at[pl.ds(base, b_per_w)], idx_v)
        pltpu.async_copy(table_hbm.at[idx_v], rows_v, sem).wait()
        pltpu.sync_copy(rows_v, zq_hbm.at[pl.ds(base, b_per_w)])
        pltpu.sync_copy(ones_hbm.at[pl.ds(base, b_per_w)], ones_v)

        @pl.when(sid == 0)
        def _():
            pltpu.sync_copy(zeros_hbm, cnt_v)
            pltpu.sync_copy(cnt_v, shared)

        plsc.subcore_barrier()
        pltpu.sync_copy(ones_v, shared.at[idx_v], add=True)
        plsc.subcore_barrier()

        @pl.when(sid == 0)
        def _():
            pltpu.sync_copy(shared, cnt_v)
            pltpu.sync_copy(cnt_v, counts_hbm.at[cid])

    return sc_gather


def _finish_body(counts_ref, lp_ref, util_ref, loss_ref, *, n_tokens):
    total = counts_ref[0, :] + counts_ref[1, :]
    util_ref[...] = jnp.sum((total > 0).astype(jnp.float32)).reshape(1, 1) * (1.0 / K)
    loss_ref[...] = (BETA * (jnp.sum(lp_ref[...]) / (n_tokens * D))).reshape(1, 1)


@jax.jit
def kernel(z_e, codebook):
    B, T, Dd = z_e.shape
    n = B * T
    z = z_e.reshape(n, Dd)
    zsq = jnp.sum(z ** 2, axis=1, keepdims=True)
    csq = jnp.sum(codebook ** 2, axis=1).reshape(1, K)
    c2 = codebook * 2.0
    grid = n // TILE_M

    k3, lp = pl.pallas_call(
        _dist_body,
        grid=(grid,),
        in_specs=[
            pl.BlockSpec((TILE_M, Dd), lambda i: (i, 0)),
            pl.BlockSpec((TILE_M, 1), lambda i: (i, 0)),
            pl.BlockSpec((K, Dd), lambda i: (0, 0)),
            pl.BlockSpec((1, K), lambda i: (0, 0)),
        ],
        out_specs=[
            pl.BlockSpec((1, 1, TILE_M), lambda i: (i, 0, 0)),
            pl.BlockSpec((1, 1, 1), lambda i: (i, 0, 0)),
        ],
        out_shape=[
            jax.ShapeDtypeStruct((grid, 1, TILE_M), jnp.int32),
            jax.ShapeDtypeStruct((grid, 1, 1), jnp.float32),
        ],
        compiler_params=pltpu.CompilerParams(
            dimension_semantics=("parallel",)),
    )(z, zsq, c2, csq)

    kflat = k3.reshape(n)
    table = jnp.pad(codebook, ((0, 0), (0, _DPAD - D)))
    zq_pad, counts2 = _make_sc_gather(n)(
        table, kflat,
        jnp.zeros((K,), jnp.int32), jnp.ones((n,), jnp.int32))
    zq = zq_pad[:, :D]

    util, loss = pl.pallas_call(
        functools.partial(_finish_body, n_tokens=n),
        out_shape=[
            jax.ShapeDtypeStruct((1, 1), jnp.float32),
            jax.ShapeDtypeStruct((1, 1), jnp.float32),
        ],
    )(counts2, lp.reshape(1, grid))

    zq_st = z + (zq - z)
    return (zq_st.reshape(B, T, Dd),
            kflat.reshape(B, T),
            loss.reshape(()),
            util.reshape(()))


# speedup vs baseline: 1.1462x; 1.1462x over previous
import functools

import jax
import jax.numpy as jnp
from jax import lax
from jax.experimental import pallas as pl
from jax.experimental.pallas import tpu as pltpu
from jax.experimental.pallas import tpu_sc as plsc

K = 8192
D = 64
BETA = 0.25
TILE_M = 512
CHUNK = 2048

_SC_INFO = plsc.get_sparse_core_info()
_NC, _NS = _SC_INFO.num_cores, _SC_INFO.num_subcores
_NW = _NC * _NS


def _dist_body(z_ref, zsq_ref, c2_ref, csq_ref, k_ref, lp_ref):
    z = z_ref[...]
    dot2 = jax.lax.dot_general(
        z, c2_ref[...], (((1,), (1,)), ((), ())),
        preferred_element_type=jnp.float32)
    d = (zsq_ref[...] - dot2) + csq_ref[...]

    nch = K // CHUNK
    acc_v = cidx = minval = None
    for t in range(nch):
        m = jnp.min(d[:, t * CHUNK:(t + 1) * CHUNK], axis=1, keepdims=True)
        if t == 0:
            minval = m
            cidx = jnp.zeros((TILE_M, 1), jnp.int32)
            acc_v = m.astype(jnp.bfloat16).astype(jnp.float32)
        else:
            upd = m < acc_v
            cidx = jnp.where(upd, t, cidx)
            minval = jnp.where(upd, m, minval)
            acc_v = jnp.where(upd, m, acc_v).astype(jnp.bfloat16).astype(jnp.float32)

    dsel = d[:, :CHUNK]
    for t in range(1, nch):
        dsel = jnp.where(cidx == t, d[:, t * CHUNK:(t + 1) * CHUNK], dsel)
    iota_l = jax.lax.broadcasted_iota(jnp.int32, (TILE_M, CHUNK), 1)
    lidx = jnp.min(jnp.where(dsel == minval, iota_l, K), axis=1)
    k_ref[0, 0, :] = lidx + cidx[:, 0] * CHUNK
    lp_ref[...] = jnp.sum(minval).reshape(1, 1, 1)


_DPAD = 128


def _make_sc_gather(n_tokens):
    b_per_w = n_tokens // _NW
    mesh = plsc.VectorSubcoreMesh(core_axis_name="c", subcore_axis_name="s")

    @functools.partial(
        pl.kernel, mesh=mesh,
        out_type=[
            jax.ShapeDtypeStruct((n_tokens, _DPAD), jnp.float32),
            jax.ShapeDtypeStruct((_NC, K), jnp.int32),
        ],
        scratch_types=[
            pltpu.VMEM((b_per_w,), jnp.int32),
            pltpu.VMEM((b_per_w, _DPAD), jnp.float32),
            pltpu.VMEM((b_per_w,), jnp.int32),
            pltpu.VMEM((K,), jnp.int32),
            pltpu.VMEM_SHARED((K,), jnp.int32),
            pltpu.SemaphoreType.DMA,
        ],
    )
    def sc_gather(table_hbm, idx_hbm, zeros_hbm, ones_hbm,
                  zq_hbm, counts_hbm,
                  idx_v, rows_v, ones_v, cnt_v, shared, sem):
        cid = lax.axis_index("c")
        sid = lax.axis_index("s")
        wid = sid * _NC + cid
        base = wid * b_per_w
        pltpu.sync_copy(idx_hbm.at[pl.ds(base, b_per_w)], idx_v)
        pltpu.async_copy(table_hbm.at[idx_v], rows_v, sem).wait()
        pltpu.sync_copy(rows_v, zq_hbm.at[pl.ds(base, b_per_w)])
        pltpu.sync_copy(ones_hbm.at[pl.ds(base, b_per_w)], ones_v)

        @pl.when(sid == 0)
        def _():
            pltpu.sync_copy(zeros_hbm, cnt_v)
            pltpu.sync_copy(cnt_v, shared)

        plsc.subcore_barrier()
        pltpu.sync_copy(ones_v, shared.at[idx_v], add=True)
        plsc.subcore_barrier()

        @pl.when(sid == 0)
        def _():
            pltpu.sync_copy(shared, cnt_v)
            pltpu.sync_copy(cnt_v, counts_hbm.at[cid])

    return sc_gather


def _finish_body(counts_ref, lp_ref, util_ref, loss_ref, *, n_tokens):
    total = counts_ref[0, :] + counts_ref[1, :]
    util_ref[...] = jnp.sum((total > 0).astype(jnp.float32)).reshape(1, 1) * (1.0 / K)
    loss_ref[...] = (BETA * (jnp.sum(lp_ref[...]) / (n_tokens * D))).reshape(1, 1)


@jax.jit
def kernel(z_e, codebook):
    B, T, Dd = z_e.shape
    n = B * T
    z = z_e.reshape(n, Dd)
    zsq = jnp.sum(z ** 2, axis=1, keepdims=True)
    csq = jnp.sum(codebook ** 2, axis=1).reshape(1, K)
    c2 = codebook * 2.0
    grid = n // TILE_M

    k3, lp = pl.pallas_call(
        _dist_body,
        grid=(grid,),
        in_specs=[
            pl.BlockSpec((TILE_M, Dd), lambda i: (i, 0)),
            pl.BlockSpec((TILE_M, 1), lambda i: (i, 0)),
            pl.BlockSpec((K, Dd), lambda i: (0, 0)),
            pl.BlockSpec((1, K), lambda i: (0, 0)),
        ],
        out_specs=[
            pl.BlockSpec((1, 1, TILE_M), lambda i: (i, 0, 0)),
            pl.BlockSpec((1, 1, 1), lambda i: (i, 0, 0)),
        ],
        out_shape=[
            jax.ShapeDtypeStruct((grid, 1, TILE_M), jnp.int32),
            jax.ShapeDtypeStruct((grid, 1, 1), jnp.float32),
        ],
        compiler_params=pltpu.CompilerParams(
            dimension_semantics=("parallel",)),
    )(z, zsq, c2, csq)

    kflat = k3.reshape(n)
    table = jnp.pad(codebook, ((0, 0), (0, _DPAD - D)))
    zq_pad, counts2 = _make_sc_gather(n)(
        table, kflat,
        jnp.zeros((K,), jnp.int32), jnp.ones((n,), jnp.int32))
    zq = zq_pad[:, :D]

    util, loss = pl.pallas_call(
        functools.partial(_finish_body, n_tokens=n),
        out_shape=[
            jax.ShapeDtypeStruct((1, 1), jnp.float32),
            jax.ShapeDtypeStruct((1, 1), jnp.float32),
        ],
    )(counts2, lp.reshape(1, grid))

    zq_st = z + (zq - z)
    return (zq_st.reshape(B, T, Dd),
            kflat.reshape(B, T),
            loss.reshape(()),
            util.reshape(()))
